# baseline (speedup 1.0000x reference)
import functools

import jax
import jax.numpy as jnp
from jax import lax
from jax.experimental import pallas as pl
from jax.experimental.pallas import tpu as pltpu
from jax.experimental.pallas import tpu_sc as plsc

BATCH = 128
N = 100000
NC = 2
NS = 16
NW = NC * NS
ROWS_PER = BATCH // NW
CHUNK = 10000
NCHUNK = N // CHUNK
NVEC = CHUNK // 16

_CONST_CACHE = {}


def _bonus():
    if "bonus" not in _CONST_CACHE:
        def mk():
            i = jnp.arange(BATCH, dtype=jnp.float32)[:, None]
            j = jnp.arange(N, dtype=jnp.float32)[None, :]
            denom = 1.0 + i * (1.0 + i * jnp.float32(N) + j)
            return jnp.float32(0.5) * jnp.sqrt(jnp.log(i + 1.0) / denom)
        _CONST_CACHE["bonus"] = jax.jit(mk)()
    return _CONST_CACHE["bonus"]


def _sc_body(probs_hbm, bonus_hbm, out_hbm, pb0, pb1, bb0, bb1, ob,
             sp0, sp1, sb0, sb1):
    cid = lax.axis_index("c")
    sid = lax.axis_index("s")
    wid = sid * NC + cid
    pbufs = (pb0, pb1)
    bbufs = (bb0, bb1)
    psems = (sp0, sp1)
    bsems = (sb0, sb1)
    lane = lax.iota(jnp.int32, 16)

    out_vec = jnp.zeros((16,), jnp.int32)
    for r in range(ROWS_PER):
        row = wid * ROWS_PER + r
        pltpu.make_async_copy(
            probs_hbm.at[row, pl.ds(0, CHUNK)], pbufs[0], psems[0]).start()
        pltpu.make_async_copy(
            bonus_hbm.at[row, pl.ds(0, CHUNK)], bbufs[0], bsems[0]).start()

        vmax = jnp.full((16,), -jnp.inf, jnp.float32)
        vidx = jnp.zeros((16,), jnp.int32)
        for k in range(NCHUNK):
            cur = k % 2
            nxt = (k + 1) % 2
            if k + 1 < NCHUNK:
                off = (k + 1) * CHUNK
                pltpu.make_async_copy(
                    probs_hbm.at[row, pl.ds(off, CHUNK)],
                    pbufs[nxt], psems[nxt]).start()
                pltpu.make_async_copy(
                    bonus_hbm.at[row, pl.ds(off, CHUNK)],
                    bbufs[nxt], bsems[nxt]).start()
            pltpu.make_async_copy(
                probs_hbm.at[row, pl.ds(k * CHUNK, CHUNK)],
                pbufs[cur], psems[cur]).wait()
            pltpu.make_async_copy(
                bonus_hbm.at[row, pl.ds(k * CHUNK, CHUNK)],
                bbufs[cur], bsems[cur]).wait()

            pb = pbufs[cur]
            bb = bbufs[cur]
            base = k * CHUNK

            def inner(v, carry, pb=pb, bb=bb, base=base):
                vm, vi = carry
                off = pl.multiple_of(v * 16, 16)
                x = pb[pl.ds(off, 16)] + bb[pl.ds(off, 16)]
                iv = lane + (base + v * 16)
                m = x > vm
                vm = jnp.where(m, x, vm)
                vi = jnp.where(m, iv, vi)
                return vm, vi

            vmax, vidx = lax.fori_loop(0, NVEC, inner, (vmax, vidx))

        maxv = jnp.max(vmax)
        cand = jnp.where(vmax == maxv, vidx, jnp.int32(2**31 - 1))
        best = jnp.min(cand)
        out_vec = jnp.where(lane == r, jnp.broadcast_to(best, (16,)), out_vec)

    ob[...] = out_vec
    pltpu.sync_copy(ob, out_hbm.at[wid])


@functools.partial(jax.jit, static_argnames=())
def _run(probs, bonus):
    fn = pl.kernel(
        _sc_body,
        out_type=jax.ShapeDtypeStruct((NW, 16), jnp.int32),
        mesh=plsc.VectorSubcoreMesh(core_axis_name="c", subcore_axis_name="s"),
        scratch_types=[
            pltpu.VMEM((CHUNK,), jnp.float32),
            pltpu.VMEM((CHUNK,), jnp.float32),
            pltpu.VMEM((CHUNK,), jnp.float32),
            pltpu.VMEM((CHUNK,), jnp.float32),
            pltpu.VMEM((16,), jnp.int32),
            pltpu.SemaphoreType.DMA,
            pltpu.SemaphoreType.DMA,
            pltpu.SemaphoreType.DMA,
            pltpu.SemaphoreType.DMA,
        ],
        compiler_params=pltpu.CompilerParams(
            use_tc_tiling_on_sc=False, needs_layout_passes=False),
    )
    return fn(probs, bonus)


def kernel(probs):
    out32 = _run(probs, _bonus())
    idx = out32[:, :ROWS_PER].reshape(BATCH)
    return idx.astype(jnp.int64)[:, None]


# speedup vs baseline: 1.2725x; 1.2725x over previous
import jax
import jax.numpy as jnp
from jax import lax
from jax.experimental import pallas as pl
from jax.experimental.pallas import tpu as pltpu
from jax.experimental.pallas import tpu_sc as plsc

BATCH = 128
N = 100000
NC = 2
NS = 16
NW = NC * NS
NHEAVY = NW
NLIGHT = (BATCH - NHEAVY) // NW
CHUNK = 20000
NCHUNK = N // CHUNK
SB = 2000
SB_PER_CHUNK = CHUNK // SB
NSB = N // SB
NACC = 5

_CONST_CACHE = {}


def _consts():
    if "bonus" not in _CONST_CACHE:
        def mk():
            i = jnp.arange(BATCH, dtype=jnp.float32)[:, None]
            j = jnp.arange(N, dtype=jnp.float32)[None, :]
            denom = 1.0 + i * (1.0 + i * jnp.float32(N) + j)
            bonus = jnp.float32(0.5) * jnp.sqrt(jnp.log(i + 1.0) / denom)
            w = (jnp.max(bonus, axis=1) - jnp.min(bonus, axis=1)
                 + jnp.float32(1e-6))
            w16 = jnp.broadcast_to(w[:, None], (BATCH, 16))
            return bonus, w16
        _CONST_CACHE["bonus"], _CONST_CACHE["w16"] = jax.jit(mk)()
    return _CONST_CACHE["bonus"], _CONST_CACHE["w16"]


def _first_index_reduce(vm, vi, lane):
    maxv = jnp.max(vm)
    cand = jnp.where(vm == maxv, vi, jnp.full((16,), 2**31 - 1, jnp.int32))
    return jnp.min(cand)


def _sc_body(probs_hbm, bonus_hbm, w_hbm, out_hbm,
             pb0, pb1, bb0, bb1, pbs, bbs, bm_ref, w_ref, ob, vacc, iacc,
             sp0, sp1, sb0, sb1, srp, srb, sw):
    cid = lax.axis_index("c")
    sid = lax.axis_index("s")
    wid = sid * NC + cid
    pbufs = (pb0, pb1)
    bbufs = (bb0, bb1)
    psems = (sp0, sp1)
    bsems = (sb0, sb1)
    lane = lax.iota(jnp.int32, 16)
    neg_inf = jnp.full((16,), -jnp.inf, jnp.float32)

    pltpu.make_async_copy(w_hbm, w_ref, sw).start()

    heavy_row = wid
    light0 = NHEAVY + wid * NLIGHT

    steps = [(heavy_row, k, True) for k in range(NCHUNK)]
    for j in range(NLIGHT):
        steps += [(light0 + j, k, False) for k in range(NCHUNK)]

    def start_step(s):
        row, k, with_bonus = steps[s]
        slot = s % 2
        pltpu.make_async_copy(
            probs_hbm.at[row, pl.ds(k * CHUNK, CHUNK)],
            pbufs[slot], psems[slot]).start()
        if with_bonus:
            pltpu.make_async_copy(
                bonus_hbm.at[row, pl.ds(k * CHUNK, CHUNK)],
                bbufs[slot], bsems[slot]).start()

    def wait_step(s):
        row, k, with_bonus = steps[s]
        slot = s % 2
        pltpu.make_async_copy(
            probs_hbm.at[row, pl.ds(k * CHUNK, CHUNK)],
            pbufs[slot], psems[slot]).wait()
        if with_bonus:
            pltpu.make_async_copy(
                bonus_hbm.at[row, pl.ds(k * CHUNK, CHUNK)],
                bbufs[slot], bsems[slot]).wait()

    start_step(0)
    pltpu.make_async_copy(w_hbm, w_ref, sw).wait()
    out_vec = jnp.zeros((16,), jnp.int32)

    accs = tuple((neg_inf, jnp.zeros((16,), jnp.int32)) for _ in range(NACC))
    for s in range(NCHUNK):
        if s + 1 < len(steps):
            start_step(s + 1)
        wait_step(s)
        slot = s % 2
        pb, bb = pbufs[slot], bbufs[slot]
        base = s * CHUNK

        def hbody(i, carry, pb=pb, bb=bb, base=base):
            new = []
            boff = i * (NACC * 16)
            for t in range(NACC):
                vm, vi = carry[2 * t], carry[2 * t + 1]
                o = pl.multiple_of(boff + t * 16, 16)
                x = pb[pl.ds(o, 16)] + bb[pl.ds(o, 16)]
                iv = lane + (base + boff + t * 16)
                m = x > vm
                new.append(jnp.where(m, x, vm))
                new.append(jnp.where(m, iv, vi))
            return tuple(new)

        flat = sum(accs, ())
        flat = lax.fori_loop(0, CHUNK // 16 // NACC, hbody, flat, unroll=2)
        accs = tuple((flat[2 * t], flat[2 * t + 1]) for t in range(NACC))

    vm, vi = accs[0]
    for t in range(1, NACC):
        bm2, bi2 = accs[t]
        take = (bm2 > vm) | ((bm2 == vm) & (bi2 < vi))
        vm = jnp.where(take, bm2, vm)
        vi = jnp.where(take, bi2, vi)
    best = _first_index_reduce(vm, vi, lane)
    out_vec = jnp.where(lane == 0, jnp.broadcast_to(best, (16,)), out_vec)

    for j in range(NLIGHT):
        row = light0 + j
        for k in range(NCHUNK):
            s = NCHUNK * (1 + j) + k
            if s + 1 < len(steps):
                start_step(s + 1)
            wait_step(s)
            pb = pbufs[s % 2]

            def sb_body(sbl, _, pb=pb, k=k):
                sb_off = sbl * SB

                def acc_body(i, acc5, pb=pb, sb_off=sb_off):
                    base = sb_off + i * (NACC * 16)
                    return tuple(
                        jnp.maximum(acc5[t], pb[pl.ds(base + t * 16, 16)])
                        for t in range(NACC))

                acc5 = lax.fori_loop(0, SB // 16 // NACC, acc_body,
                                     (neg_inf,) * NACC, unroll=5)
                m = jnp.maximum(jnp.maximum(acc5[0], acc5[1]),
                                jnp.maximum(acc5[2], acc5[3]))
                m = jnp.maximum(m, acc5[4])
                bm_off = (k * SB_PER_CHUNK + sbl) * 16
                bm_ref[pl.ds(bm_off, 16)] = m
                return 0

            lax.fori_loop(0, SB_PER_CHUNK, sb_body, 0)

        def rmax_body(sb, acc):
            return jnp.maximum(acc, bm_ref[pl.ds(sb * 16, 16)])

        rowmax = jnp.max(lax.fori_loop(0, NSB, rmax_body, neg_inf, unroll=5))
        wv = w_ref[pl.ds(pl.multiple_of(row * 16, 16), 16)]
        thr = jnp.broadcast_to(rowmax, (16,)) - wv

        vacc[...] = neg_inf
        iacc[...] = jnp.zeros((16,), jnp.int32)

        def cand_body(sb, _, row=row):
            bmv = bm_ref[pl.ds(sb * 16, 16)]
            qual = jnp.any(bmv >= thr)

            @pl.when(qual)
            def taken(sb=sb, row=row):
                off = pl.multiple_of(sb * SB, 8)
                cp = pltpu.make_async_copy(
                    probs_hbm.at[row, pl.ds(off, SB)], pbs, srp)
                cb = pltpu.make_async_copy(
                    bonus_hbm.at[row, pl.ds(off, SB)], bbs, srb)
                cp.start()
                cb.start()
                cp.wait()
                cb.wait()

                def scan_body(v, c2, sb=sb):
                    vm, vi = c2
                    o = pl.multiple_of(v * 16, 16)
                    x = pbs[pl.ds(o, 16)] + bbs[pl.ds(o, 16)]
                    iv = lane + (sb * SB + v * 16)
                    m = x > vm
                    return jnp.where(m, x, vm), jnp.where(m, iv, vi)

                vm, vi = lax.fori_loop(0, SB // 16, scan_body,
                                       (vacc[...], iacc[...]), unroll=5)
                vacc[...] = vm
                iacc[...] = vi

            return 0

        lax.fori_loop(0, NSB, cand_body, 0)
        best = _first_index_reduce(vacc[...], iacc[...], lane)
        out_vec = jnp.where(lane == (1 + j), jnp.broadcast_to(best, (16,)),
                            out_vec)

    ob[...] = out_vec
    pltpu.sync_copy(ob, out_hbm.at[wid])


@jax.jit
def _run(probs, bonus, w16):
    fn = pl.kernel(
        _sc_body,
        out_type=jax.ShapeDtypeStruct((NW, 16), jnp.int32),
        mesh=plsc.VectorSubcoreMesh(core_axis_name="c", subcore_axis_name="s"),
        scratch_types=[
            pltpu.VMEM((CHUNK,), jnp.float32),
            pltpu.VMEM((CHUNK,), jnp.float32),
            pltpu.VMEM((CHUNK,), jnp.float32),
            pltpu.VMEM((CHUNK,), jnp.float32),
            pltpu.VMEM((SB,), jnp.float32),
            pltpu.VMEM((SB,), jnp.float32),
            pltpu.VMEM((NSB * 16,), jnp.float32),
            pltpu.VMEM((BATCH * 16,), jnp.float32),
            pltpu.VMEM((16,), jnp.int32),
            pltpu.VMEM((16,), jnp.float32),
            pltpu.VMEM((16,), jnp.int32),
            pltpu.SemaphoreType.DMA,
            pltpu.SemaphoreType.DMA,
            pltpu.SemaphoreType.DMA,
            pltpu.SemaphoreType.DMA,
            pltpu.SemaphoreType.DMA,
            pltpu.SemaphoreType.DMA,
            pltpu.SemaphoreType.DMA,
        ],
        compiler_params=pltpu.CompilerParams(
            use_tc_tiling_on_sc=False, needs_layout_passes=False),
    )
    return fn(probs, bonus, w16.reshape(BATCH * 16))


def kernel(probs):
    bonus, w16 = _consts()
    out32 = _run(probs, bonus, w16)
    heavy = out32[:, 0]
    light = out32[:, 1:1 + NLIGHT].reshape(NW * NLIGHT)
    idx = jnp.concatenate([heavy, light])
    return idx.astype(jnp.int64)[:, None]


# speedup vs baseline: 3.3377x; 2.6229x over previous
import jax
import jax.numpy as jnp
from jax import lax
from jax.experimental import pallas as pl
from jax.experimental.pallas import tpu as pltpu
from jax.experimental.pallas import tpu_sc as plsc

BATCH = 128
N = 100000
NC = 2
NS = 16
NW = NC * NS
SHARD = N // NW
CCOLS = 125
NCHUNK = SHARD // CCOLS
NG = BATCH // 16

_CONST_CACHE = {}


def _bonus_t():
    if "bonus_t" not in _CONST_CACHE:
        def mk():
            i = jnp.arange(BATCH, dtype=jnp.float32)[None, :]
            j = jnp.arange(N, dtype=jnp.float32)[:, None]
            denom = 1.0 + i * (1.0 + i * jnp.float32(N) + j)
            return jnp.float32(0.5) * jnp.sqrt(jnp.log(i + 1.0) / denom)
        _CONST_CACHE["bonus_t"] = jax.jit(mk)()
    return _CONST_CACHE["bonus_t"]


def _sc_body(pt_hbm, bt_hbm, val_hbm, idx_hbm,
             pb0, pb1, bb0, bb1, ov, oi, sp0, sp1, sb0, sb1):
    cid = lax.axis_index("c")
    sid = lax.axis_index("s")
    wid = sid * NC + cid
    pbufs = (pb0, pb1)
    bbufs = (bb0, bb1)
    psems = (sp0, sp1)
    bsems = (sb0, sb1)
    col0 = wid * SHARD
    neg_inf = jnp.full((16,), -jnp.inf, jnp.float32)

    def start_chunk(k, slot):
        pltpu.make_async_copy(
            pt_hbm.at[pl.ds(col0 + k * CCOLS, CCOLS)],
            pbufs[slot], psems[slot]).start()
        pltpu.make_async_copy(
            bt_hbm.at[pl.ds(col0 + k * CCOLS, CCOLS)],
            bbufs[slot], bsems[slot]).start()

    def wait_chunk(k, slot):
        pltpu.make_async_copy(
            pt_hbm.at[pl.ds(col0 + k * CCOLS, CCOLS)],
            pbufs[slot], psems[slot]).wait()
        pltpu.make_async_copy(
            bt_hbm.at[pl.ds(col0 + k * CCOLS, CCOLS)],
            bbufs[slot], bsems[slot]).wait()

    start_chunk(0, 0)
    accs = []
    for b in range(NG):
        accs.append(neg_inf)
        accs.append(jnp.zeros((16,), jnp.int32))
    accs = tuple(accs)

    for k in range(NCHUNK):
        slot = k % 2
        if k + 1 < NCHUNK:
            start_chunk(k + 1, (k + 1) % 2)
        wait_chunk(k, slot)
        pb, bb = pbufs[slot], bbufs[slot]
        cbase = col0 + k * CCOLS

        def cbody(v, carry, pb=pb, bb=bb, cbase=cbase):
            iv = jnp.broadcast_to(cbase + v, (16,))
            new = []
            for b in range(NG):
                vm, vi = carry[2 * b], carry[2 * b + 1]
                o = 16 * b
                x = pb[v, pl.ds(o, 16)] + bb[v, pl.ds(o, 16)]
                m = x > vm
                new.append(jnp.where(m, x, vm))
                new.append(jnp.where(m, iv, vi))
            return tuple(new)

        accs = lax.fori_loop(0, CCOLS, cbody, accs)

    for b in range(NG):
        ov[pl.ds(16 * b, 16)] = accs[2 * b]
        oi[pl.ds(16 * b, 16)] = accs[2 * b + 1]
    pltpu.sync_copy(ov, val_hbm.at[wid])
    pltpu.sync_copy(oi, idx_hbm.at[wid])


@jax.jit
def _run(pt, bt):
    fn = pl.kernel(
        _sc_body,
        out_type=(
            jax.ShapeDtypeStruct((NW, BATCH), jnp.float32),
            jax.ShapeDtypeStruct((NW, BATCH), jnp.int32),
        ),
        mesh=plsc.VectorSubcoreMesh(core_axis_name="c", subcore_axis_name="s"),
        scratch_types=[
            pltpu.VMEM((CCOLS, BATCH), jnp.float32),
            pltpu.VMEM((CCOLS, BATCH), jnp.float32),
            pltpu.VMEM((CCOLS, BATCH), jnp.float32),
            pltpu.VMEM((CCOLS, BATCH), jnp.float32),
            pltpu.VMEM((BATCH,), jnp.float32),
            pltpu.VMEM((BATCH,), jnp.int32),
            pltpu.SemaphoreType.DMA,
            pltpu.SemaphoreType.DMA,
            pltpu.SemaphoreType.DMA,
            pltpu.SemaphoreType.DMA,
        ],
        compiler_params=pltpu.CompilerParams(
            use_tc_tiling_on_sc=False, needs_layout_passes=False),
    )
    vals, idxs = fn(pt, bt)
    best = jnp.max(vals, axis=0, keepdims=True)
    cand = jnp.where(vals == best, idxs, jnp.int32(2**31 - 1))
    return jnp.min(cand, axis=0)


def kernel(probs):
    idx = _run(probs.T, _bonus_t())
    return idx.astype(jnp.int64)[:, None]
